# Initial kernel scaffold; baseline (speedup 1.0000x reference)
import jax
import jax.numpy as jnp
from jax.experimental import pallas as pl


def kernel(x, pos_edge_index, neg_edge_index, W1_b, b1_b, att1_b, W1_u, b1_u, att1_u, W2_b, b2_b, att2_bp, att2_bn, W2_u, b2_u, att2_up, att2_un, Wf, bf):
    raise NotImplementedError("write your pallas kernel here")


# speedup vs baseline: 2.4577x; 2.4577x over previous
import jax
import jax.numpy as jnp
from jax import lax
from jax.experimental import pallas as pl
from jax.experimental.pallas import tpu as pltpu
from jax.experimental.pallas import tpu_sc as plsc

N = 10000
NP = 10240
D = 256
HID = 128
E = 80000
NC, NS, L = 2, 16, 16
NW = NC * NS
CH = 128
EPW = 2560
NCHUNK = EPW // CH
EP = NW * EPW
RPT = NP // NS
EPS = 1e-16


HALF = NP // 2
RPH = HALF // NS
DUMMY = HALF - 1


def _sc_body(feat, src, dst, ad, asr, acc_out, den_out,
             ad_v, as_v, den_v, src_v, dst_v,
             srcA, dstA, scoA, srcB, dstB, scoB, grows, srows, acc, sem):
    c = lax.axis_index("c")
    s = lax.axis_index("s")
    wid = s * NC + c

    pltpu.sync_copy(src.at[wid], src_v)
    pltpu.sync_copy(dst.at[wid], dst_v)
    pltpu.sync_copy(ad, ad_v)
    pltpu.sync_copy(asr, as_v)

    zero = jnp.zeros((L,), jnp.float32)
    zeroi = jnp.zeros((L,), jnp.int32)
    dum = jnp.full((L,), DUMMY, jnp.int32)

    def zden(g, carry):
        den_v[pl.ds(g * L, L)] = zero
        return carry

    lax.fori_loop(0, NP // L, zden, 0)

    def ibkt(j, carry):
        for k in range(CH // L):
            srcA[j, pl.ds(k * L, L)] = zeroi
            srcB[j, pl.ds(k * L, L)] = zeroi
            dstA[j, pl.ds(k * L, L)] = dum
            dstB[j, pl.ds(k * L, L)] = dum
            scoA[j, pl.ds(k * L, L)] = zero
            scoB[j, pl.ds(k * L, L)] = zero
        return carry

    lax.fori_loop(0, NCHUNK, ibkt, 0)

    def binb(g, carry):
        cA, cB = carry
        de = dst_v[pl.ds(g * L, L)]
        se = src_v[pl.ds(g * L, L)]
        t = plsc.load_gather(ad_v, [de]) + plsc.load_gather(as_v, [se])
        t = jnp.maximum(t, 0.2 * t)
        sco = jnp.exp(t)
        m = de < HALF
        mi = m.astype(jnp.int32)
        nA = jnp.sum(mi)
        posA = cA + plsc.cumsum(mi) - 1
        mB = jnp.logical_not(m)
        posB = cB + plsc.cumsum(mB.astype(jnp.int32)) - 1
        rA, qA = posA >> 7, posA & (CH - 1)
        rB, qB = posB >> 7, posB & (CH - 1)
        plsc.store_scatter(srcA, [rA, qA], se, mask=m)
        plsc.store_scatter(dstA, [rA, qA], de, mask=m)
        plsc.store_scatter(scoA, [rA, qA], sco, mask=m)
        plsc.store_scatter(srcB, [rB, qB], se, mask=mB)
        plsc.store_scatter(dstB, [rB, qB], de - HALF, mask=mB)
        plsc.store_scatter(scoB, [rB, qB], sco, mask=mB)
        return cA + nA, cB + (L - nA)

    cA, cB = lax.fori_loop(0, EPW // L, binb, (jnp.int32(0), jnp.int32(0)))

    base = s * RPH
    for off, sb_, db_, scb_, cnt in ((0, srcA, dstA, scoA, cA),
                                     (HALF, srcB, dstB, scoB, cB)):
        def zs(r, carry):
            for k in range(HID // L):
                srows[r, pl.ds(k * L, L)] = zero
            return carry

        lax.fori_loop(0, CH, zs, 0)
        pltpu.sync_copy(srows, acc.at[pl.ds(base, CH)])
        pltpu.sync_copy(srows, acc.at[pl.ds(base + CH, CH)])
        pltpu.sync_copy(srows.at[pl.ds(0, RPH - 2 * CH)],
                        acc.at[pl.ds(base + 2 * CH, RPH - 2 * CH)])
        plsc.subcore_barrier()

        def chunk(j, carry):
            pltpu.async_copy(feat.at[sb_.at[j]], grows, sem).wait()

            def row(r, carry2):
                r16 = jnp.full((L,), r, jnp.int32)
                j16 = jnp.full((L,), j, jnp.int32)
                sv = plsc.load_gather(scb_, [j16, r16])
                for k in range(HID // L):
                    srows[r, pl.ds(k * L, L)] = grows[r, pl.ds(k * L, L)] * sv
                di = plsc.load_gather(db_, [j16, r16]) + off
                cur = plsc.load_gather(den_v, [di])
                plsc.store_scatter(den_v, [di], cur + sv)
                return carry2

            lax.fori_loop(0, CH, row, 0)
            pltpu.sync_copy(srows, acc.at[db_.at[j]], add=True)
            return carry

        lax.fori_loop(0, (cnt + CH - 1) >> 7, chunk, 0)
        plsc.subcore_barrier()

        pltpu.sync_copy(acc.at[pl.ds(base, RPH)],
                        acc_out.at[c, pl.ds(off + base, RPH)])

    pltpu.sync_copy(den_v, den_out.at[c, s])


_sc_agg = pl.kernel(
    _sc_body,
    out_type=(jax.ShapeDtypeStruct((NC, NP, HID), jnp.float32),
              jax.ShapeDtypeStruct((NC, NS, NP), jnp.float32)),
    mesh=plsc.VectorSubcoreMesh(core_axis_name="c", subcore_axis_name="s",
                                num_cores=NC, num_subcores=NS),
    scratch_types=[
        pltpu.VMEM((NP,), jnp.float32),
        pltpu.VMEM((NP,), jnp.float32),
        pltpu.VMEM((NP,), jnp.float32),
        pltpu.VMEM((EPW,), jnp.int32),
        pltpu.VMEM((EPW,), jnp.int32),
        pltpu.VMEM((NCHUNK, CH), jnp.int32),
        pltpu.VMEM((NCHUNK, CH), jnp.int32),
        pltpu.VMEM((NCHUNK, CH), jnp.float32),
        pltpu.VMEM((NCHUNK, CH), jnp.int32),
        pltpu.VMEM((NCHUNK, CH), jnp.int32),
        pltpu.VMEM((NCHUNK, CH), jnp.float32),
        pltpu.VMEM((CH, HID), jnp.float32),
        pltpu.VMEM((CH, HID), jnp.float32),
        pltpu.VMEM_SHARED((HALF, HID), jnp.float32),
        pltpu.SemaphoreType.DMA,
    ],
    compiler_params=pltpu.CompilerParams(needs_layout_passes=False),
    name="snea_sc_agg",
)


BLK = 512


def _mm_body(x_ref, a_ref, o_ref):
    o_ref[...] = jnp.dot(x_ref[...], a_ref[...],
                         preferred_element_type=jnp.float32)


def _alpha1(x_pad, a1):
    return pl.pallas_call(
        _mm_body,
        grid=(NP // BLK,),
        in_specs=[
            pl.BlockSpec((BLK, D), lambda i: (i, 0)),
            pl.BlockSpec((D, 128), lambda i: (0, 0)),
        ],
        out_specs=pl.BlockSpec((BLK, 128), lambda i: (i, 0)),
        out_shape=jax.ShapeDtypeStruct((NP, 128), jnp.float32),
    )(x_pad, a1)


def _norm(acc, den):
    tot = acc[0] + acc[1]
    d = jnp.sum(den, axis=(0, 1))
    return tot * (1.0 / (d + EPS))[:, None]


def _tck1_body(pb0, pb1, nu0, nu1, pden, nden, x_ref,
               w1b, b1b, w1u, b1u, a2b, a2u, zb_o, zu_o, al_o):
    pd = pden[...]
    nd = nden[...]
    h0 = _norm(pb0[...], pd)
    h1 = _norm(pb1[...], pd)
    g0 = _norm(nu0[...], nd)
    g1 = _norm(nu1[...], nd)
    xb = x_ref[...]
    ob = (jnp.dot(h0, w1b[0:HID], preferred_element_type=jnp.float32)
          + jnp.dot(h1, w1b[HID:D], preferred_element_type=jnp.float32)
          + jnp.dot(xb, w1b[D:2 * D], preferred_element_type=jnp.float32)
          + b1b[...])
    ou = (jnp.dot(g0, w1u[0:HID], preferred_element_type=jnp.float32)
          + jnp.dot(g1, w1u[HID:D], preferred_element_type=jnp.float32)
          + jnp.dot(xb, w1u[D:2 * D], preferred_element_type=jnp.float32)
          + b1u[...])
    zb = jnp.tanh(ob)
    zu = jnp.tanh(ou)
    zb_o[...] = zb
    zu_o[...] = zu
    al_o[...] = (jnp.dot(zb, a2b[...], preferred_element_type=jnp.float32)
                 + jnp.dot(zu, a2u[...], preferred_element_type=jnp.float32))


def _tck1(pb0, pb1, nu0, nu1, pden, nden, x_pad, w1bt, b1b, w1ut, b1u,
          a2b, a2u):
    acc_spec = pl.BlockSpec((NC, BLK, HID), lambda i: (0, i, 0))
    den_spec = pl.BlockSpec((NC, NS, BLK), lambda i: (0, 0, i))
    w_spec = pl.BlockSpec((2 * D, HID), lambda i: (0, 0))
    b_spec = pl.BlockSpec((1, HID), lambda i: (0, 0))
    a_spec = pl.BlockSpec((HID, 128), lambda i: (0, 0))
    o_spec = pl.BlockSpec((BLK, 128), lambda i: (i, 0))
    return pl.pallas_call(
        _tck1_body,
        grid=(NP // BLK,),
        in_specs=[acc_spec, acc_spec, acc_spec, acc_spec, den_spec, den_spec,
                  pl.BlockSpec((BLK, D), lambda i: (i, 0)),
                  w_spec, b_spec, w_spec, b_spec, a_spec, a_spec],
        out_specs=[o_spec, o_spec, o_spec],
        out_shape=[jax.ShapeDtypeStruct((NP, 128), jnp.float32)] * 3,
    )(pb0, pb1, nu0, nu1, pden, nden, x_pad, w1bt, b1b, w1ut, b1u, a2b, a2u)


def _tck2_body(bp, bn, un, up, bpd, bnd, und, upd, zb_ref, zu_ref,
               w2b, b2b, w2u, b2u, wf, bf, o_ref):
    hbp = _norm(bp[...], bpd[...])
    hbn = _norm(bn[...], bnd[...])
    hun = _norm(un[...], und[...])
    hup = _norm(up[...], upd[...])
    zb = zb_ref[...]
    zu = zu_ref[...]
    ob = (jnp.dot(hbp, w2b[0:HID], preferred_element_type=jnp.float32)
          + jnp.dot(hbn, w2b[HID:2 * HID], preferred_element_type=jnp.float32)
          + jnp.dot(zb, w2b[2 * HID:3 * HID],
                    preferred_element_type=jnp.float32)
          + b2b[...])
    ou = (jnp.dot(hup, w2u[0:HID], preferred_element_type=jnp.float32)
          + jnp.dot(hun, w2u[HID:2 * HID], preferred_element_type=jnp.float32)
          + jnp.dot(zu, w2u[2 * HID:3 * HID],
                    preferred_element_type=jnp.float32)
          + b2u[...])
    z2b = jnp.tanh(ob)
    z2u = jnp.tanh(ou)
    o_ref[...] = jnp.tanh(
        jnp.dot(z2b, wf[0:HID], preferred_element_type=jnp.float32)
        + jnp.dot(z2u, wf[HID:D], preferred_element_type=jnp.float32)
        + bf[...])


def _tck2(bp, bn, un, up, bpd, bnd, und, upd, zb, zu,
          w2bt, b2b, w2ut, b2u, wft, bf):
    acc_spec = pl.BlockSpec((NC, BLK, HID), lambda i: (0, i, 0))
    den_spec = pl.BlockSpec((NC, NS, BLK), lambda i: (0, 0, i))
    z_spec = pl.BlockSpec((BLK, HID), lambda i: (i, 0))
    w_spec = pl.BlockSpec((3 * HID, HID), lambda i: (0, 0))
    b_spec = pl.BlockSpec((1, HID), lambda i: (0, 0))
    return pl.pallas_call(
        _tck2_body,
        grid=(NP // BLK,),
        in_specs=[acc_spec, acc_spec, acc_spec, acc_spec,
                  den_spec, den_spec, den_spec, den_spec, z_spec, z_spec,
                  w_spec, b_spec, w_spec, b_spec,
                  pl.BlockSpec((D, D), lambda i: (0, 0)),
                  pl.BlockSpec((1, D), lambda i: (0, 0))],
        out_specs=pl.BlockSpec((BLK, D), lambda i: (i, 0)),
        out_shape=jax.ShapeDtypeStruct((NP, D), jnp.float32),
    )(bp, bn, un, up, bpd, bnd, und, upd, zb, zu,
      w2bt, b2b, w2ut, b2u, wft, bf)


def _pad_edges(ei):
    s = ei[0].astype(jnp.int32)
    d = ei[1].astype(jnp.int32)
    sp = jnp.concatenate([s, jnp.zeros((EP - E,), jnp.int32)])
    dp = jnp.concatenate([d, jnp.full((EP - E,), N, jnp.int32)])
    return sp.reshape(NW, EPW), dp.reshape(NW, EPW)


def kernel(x, pos_edge_index, neg_edge_index, W1_b, b1_b, att1_b, W1_u, b1_u,
           att1_u, W2_b, b2_b, att2_bp, att2_bn, W2_u, b2_u, att2_up,
           att2_un, Wf, bf):
    f32 = jnp.float32
    x = x.astype(f32)
    spp, dpp = _pad_edges(pos_edge_index)
    snn, dnn = _pad_edges(neg_edge_index)
    x_pad = jnp.pad(x, ((0, NP - N), (0, 0)))
    x0 = x_pad[:, :HID]
    x1 = x_pad[:, HID:]

    a1 = jnp.zeros((D, 128), f32)
    a1 = a1.at[:, 0].set(att1_b[:D]).at[:, 1].set(att1_b[D:])
    a1 = a1.at[:, 2].set(att1_u[:D]).at[:, 3].set(att1_u[D:])
    al1 = _alpha1(x_pad, a1)

    pb0, pden = _sc_agg(x0, spp, dpp, al1[:, 0], al1[:, 1])
    pb1, _ = _sc_agg(x1, spp, dpp, al1[:, 0], al1[:, 1])
    nu0, nden = _sc_agg(x0, snn, dnn, al1[:, 2], al1[:, 3])
    nu1, _ = _sc_agg(x1, snn, dnn, al1[:, 2], al1[:, 3])

    a2b = jnp.zeros((HID, 128), f32)
    a2b = a2b.at[:, 0].set(att2_bp[:HID]).at[:, 1].set(att2_bp[HID:])
    a2b = a2b.at[:, 2].set(att2_un[:HID]).at[:, 3].set(att2_un[HID:])
    a2u = jnp.zeros((HID, 128), f32)
    a2u = a2u.at[:, 4].set(att2_up[:HID]).at[:, 5].set(att2_up[HID:])
    a2u = a2u.at[:, 6].set(att2_bn[:HID]).at[:, 7].set(att2_bn[HID:])

    zb, zu, al2 = _tck1(pb0, pb1, nu0, nu1, pden, nden, x_pad,
                        W1_b.T, b1_b.reshape(1, HID),
                        W1_u.T, b1_u.reshape(1, HID), a2b, a2u)

    bp, bpd = _sc_agg(zb, spp, dpp, al2[:, 0], al2[:, 1])
    un, und = _sc_agg(zb, snn, dnn, al2[:, 2], al2[:, 3])
    up, upd = _sc_agg(zu, spp, dpp, al2[:, 4], al2[:, 5])
    bn, bnd = _sc_agg(zu, snn, dnn, al2[:, 6], al2[:, 7])

    out = _tck2(bp, bn, un, up, bpd, bnd, und, upd, zb, zu,
                W2_b.T, b2_b.reshape(1, HID),
                W2_u.T, b2_u.reshape(1, HID),
                Wf.T, bf.reshape(1, D))
    return out[:N]


# speedup vs baseline: 2.4839x; 1.0106x over previous
import jax
import jax.numpy as jnp
from jax import lax
from jax.experimental import pallas as pl
from jax.experimental.pallas import tpu as pltpu
from jax.experimental.pallas import tpu_sc as plsc

N = 10000
NP = 10240
D = 256
HID = 128
E = 80000
NC, NS, L = 2, 16, 16
NW = NC * NS
CH = 128
EPW = 2560
NCHUNK = EPW // CH
EP = NW * EPW
RPT = NP // NS
EPS = 1e-16


HALF = NP // 2
RPH = HALF // NS
DUMMY = HALF - 1


def _sc_body(feat, src, dst, ad, asr, acc_out, den_out,
             ad_v, as_v, den_v, src_v, dst_v,
             srcA, dstA, scoA, srcB, dstB, scoB,
             grows0, grows1, acc, semg0, semg1, sems0, sems1):
    c = lax.axis_index("c")
    s = lax.axis_index("s")
    wid = s * NC + c

    pltpu.sync_copy(src.at[wid], src_v)
    pltpu.sync_copy(dst.at[wid], dst_v)
    pltpu.sync_copy(ad, ad_v)
    pltpu.sync_copy(asr, as_v)

    zero = jnp.zeros((L,), jnp.float32)
    zeroi = jnp.zeros((L,), jnp.int32)
    dum = jnp.full((L,), DUMMY, jnp.int32)

    def zden(g, carry):
        den_v[pl.ds(g * L, L)] = zero
        return carry

    lax.fori_loop(0, NP // L, zden, 0, unroll=8)

    def ibkt(j, carry):
        for k in range(CH // L):
            srcA[j, pl.ds(k * L, L)] = zeroi
            srcB[j, pl.ds(k * L, L)] = zeroi
            dstA[j, pl.ds(k * L, L)] = dum
            dstB[j, pl.ds(k * L, L)] = dum
            scoA[j, pl.ds(k * L, L)] = zero
            scoB[j, pl.ds(k * L, L)] = zero
        return carry

    lax.fori_loop(0, NCHUNK, ibkt, 0)

    def binb(g, carry):
        cA, cB = carry
        de = dst_v[pl.ds(g * L, L)]
        se = src_v[pl.ds(g * L, L)]
        t = plsc.load_gather(ad_v, [de]) + plsc.load_gather(as_v, [se])
        t = jnp.maximum(t, 0.2 * t)
        sco = jnp.exp(t)
        plsc.addupdate_scatter(den_v, [de], sco)
        m = de < HALF
        mi = m.astype(jnp.int32)
        nA = jnp.sum(mi)
        posA = cA + plsc.cumsum(mi) - 1
        mB = jnp.logical_not(m)
        posB = cB + plsc.cumsum(mB.astype(jnp.int32)) - 1
        rA, qA = posA >> 7, posA & (CH - 1)
        rB, qB = posB >> 7, posB & (CH - 1)
        plsc.store_scatter(srcA, [rA, qA], se, mask=m)
        plsc.store_scatter(dstA, [rA, qA], de, mask=m)
        plsc.store_scatter(scoA, [rA, qA], sco, mask=m)
        plsc.store_scatter(srcB, [rB, qB], se, mask=mB)
        plsc.store_scatter(dstB, [rB, qB], de - HALF, mask=mB)
        plsc.store_scatter(scoB, [rB, qB], sco, mask=mB)
        return cA + nA, cB + (L - nA)

    cA, cB = lax.fori_loop(0, EPW // L, binb, (jnp.int32(0), jnp.int32(0)))

    base = s * RPH
    for off, sb_, db_, scb_, cnt in ((0, srcA, dstA, scoA, cA),
                                     (HALF, srcB, dstB, scoB, cB)):
        def zs(r, carry):
            for k in range(HID // L):
                grows0[r, pl.ds(k * L, L)] = zero
            return carry

        lax.fori_loop(0, CH, zs, 0, unroll=4)
        pltpu.sync_copy(grows0, acc.at[pl.ds(base, CH)])
        pltpu.sync_copy(grows0, acc.at[pl.ds(base + CH, CH)])
        pltpu.sync_copy(grows0.at[pl.ds(0, RPH - 2 * CH)],
                        acc.at[pl.ds(base + 2 * CH, RPH - 2 * CH)])
        plsc.subcore_barrier()

        def scale(j, gb):
            def row(r, carry2):
                r16 = jnp.full((L,), r, jnp.int32)
                j16 = jnp.full((L,), j, jnp.int32)
                sv = plsc.load_gather(scb_, [j16, r16])
                for k in range(HID // L):
                    gb[r, pl.ds(k * L, L)] = gb[r, pl.ds(k * L, L)] * sv
                return carry2

            lax.fori_loop(0, CH, row, 0, unroll=4)

        def pair(t, carry):
            j0 = 2 * t
            j1 = j0 + 1
            g0 = pltpu.async_copy(feat.at[sb_.at[j0]], grows0, semg0)
            g1 = pltpu.async_copy(feat.at[sb_.at[j1]], grows1, semg1)
            g0.wait()
            scale(j0, grows0)
            s0 = pltpu.async_copy(grows0, acc.at[db_.at[j0]], sems0, add=True)
            g1.wait()
            scale(j1, grows1)
            s1 = pltpu.async_copy(grows1, acc.at[db_.at[j1]], sems1, add=True)
            s0.wait()
            s1.wait()
            return carry

        lax.fori_loop(0, (cnt + 2 * CH - 1) >> 8, pair, 0)
        plsc.subcore_barrier()

        pltpu.sync_copy(acc.at[pl.ds(base, RPH)],
                        acc_out.at[c, pl.ds(off + base, RPH)])

    pltpu.sync_copy(den_v, den_out.at[c, s])


_sc_agg = pl.kernel(
    _sc_body,
    out_type=(jax.ShapeDtypeStruct((NC, NP, HID), jnp.float32),
              jax.ShapeDtypeStruct((NC, NS, NP), jnp.float32)),
    mesh=plsc.VectorSubcoreMesh(core_axis_name="c", subcore_axis_name="s",
                                num_cores=NC, num_subcores=NS),
    scratch_types=[
        pltpu.VMEM((NP,), jnp.float32),
        pltpu.VMEM((NP,), jnp.float32),
        pltpu.VMEM((NP,), jnp.float32),
        pltpu.VMEM((EPW,), jnp.int32),
        pltpu.VMEM((EPW,), jnp.int32),
        pltpu.VMEM((NCHUNK, CH), jnp.int32),
        pltpu.VMEM((NCHUNK, CH), jnp.int32),
        pltpu.VMEM((NCHUNK, CH), jnp.float32),
        pltpu.VMEM((NCHUNK, CH), jnp.int32),
        pltpu.VMEM((NCHUNK, CH), jnp.int32),
        pltpu.VMEM((NCHUNK, CH), jnp.float32),
        pltpu.VMEM((CH, HID), jnp.float32),
        pltpu.VMEM((CH, HID), jnp.float32),
        pltpu.VMEM_SHARED((HALF, HID), jnp.float32),
        pltpu.SemaphoreType.DMA,
        pltpu.SemaphoreType.DMA,
        pltpu.SemaphoreType.DMA,
        pltpu.SemaphoreType.DMA,
    ],
    compiler_params=pltpu.CompilerParams(needs_layout_passes=False),
    name="snea_sc_agg",
)


BLK = 512


def _mm_body(x_ref, a_ref, o_ref):
    o_ref[...] = jnp.dot(x_ref[...], a_ref[...],
                         preferred_element_type=jnp.float32)


def _alpha1(x_pad, a1):
    return pl.pallas_call(
        _mm_body,
        grid=(NP // BLK,),
        in_specs=[
            pl.BlockSpec((BLK, D), lambda i: (i, 0)),
            pl.BlockSpec((D, 128), lambda i: (0, 0)),
        ],
        out_specs=pl.BlockSpec((BLK, 128), lambda i: (i, 0)),
        out_shape=jax.ShapeDtypeStruct((NP, 128), jnp.float32),
    )(x_pad, a1)


def _norm(acc, den):
    tot = acc[0] + acc[1]
    d = jnp.sum(den, axis=(0, 1))
    return tot * (1.0 / (d + EPS))[:, None]


def _tck1_body(pb0, pb1, nu0, nu1, pden, nden, x_ref,
               w1b, b1b, w1u, b1u, a2b, a2u, zb_o, zu_o, al_o):
    pd = pden[...]
    nd = nden[...]
    h0 = _norm(pb0[...], pd)
    h1 = _norm(pb1[...], pd)
    g0 = _norm(nu0[...], nd)
    g1 = _norm(nu1[...], nd)
    xb = x_ref[...]
    ob = (jnp.dot(h0, w1b[0:HID], preferred_element_type=jnp.float32)
          + jnp.dot(h1, w1b[HID:D], preferred_element_type=jnp.float32)
          + jnp.dot(xb, w1b[D:2 * D], preferred_element_type=jnp.float32)
          + b1b[...])
    ou = (jnp.dot(g0, w1u[0:HID], preferred_element_type=jnp.float32)
          + jnp.dot(g1, w1u[HID:D], preferred_element_type=jnp.float32)
          + jnp.dot(xb, w1u[D:2 * D], preferred_element_type=jnp.float32)
          + b1u[...])
    zb = jnp.tanh(ob)
    zu = jnp.tanh(ou)
    zb_o[...] = zb
    zu_o[...] = zu
    al_o[...] = (jnp.dot(zb, a2b[...], preferred_element_type=jnp.float32)
                 + jnp.dot(zu, a2u[...], preferred_element_type=jnp.float32))


def _tck1(pb0, pb1, nu0, nu1, pden, nden, x_pad, w1bt, b1b, w1ut, b1u,
          a2b, a2u):
    acc_spec = pl.BlockSpec((NC, BLK, HID), lambda i: (0, i, 0))
    den_spec = pl.BlockSpec((NC, NS, BLK), lambda i: (0, 0, i))
    w_spec = pl.BlockSpec((2 * D, HID), lambda i: (0, 0))
    b_spec = pl.BlockSpec((1, HID), lambda i: (0, 0))
    a_spec = pl.BlockSpec((HID, 128), lambda i: (0, 0))
    o_spec = pl.BlockSpec((BLK, 128), lambda i: (i, 0))
    return pl.pallas_call(
        _tck1_body,
        grid=(NP // BLK,),
        in_specs=[acc_spec, acc_spec, acc_spec, acc_spec, den_spec, den_spec,
                  pl.BlockSpec((BLK, D), lambda i: (i, 0)),
                  w_spec, b_spec, w_spec, b_spec, a_spec, a_spec],
        out_specs=[o_spec, o_spec, o_spec],
        out_shape=[jax.ShapeDtypeStruct((NP, 128), jnp.float32)] * 3,
    )(pb0, pb1, nu0, nu1, pden, nden, x_pad, w1bt, b1b, w1ut, b1u, a2b, a2u)


def _tck2_body(bp, bn, un, up, bpd, bnd, und, upd, zb_ref, zu_ref,
               w2b, b2b, w2u, b2u, wf, bf, o_ref):
    hbp = _norm(bp[...], bpd[...])
    hbn = _norm(bn[...], bnd[...])
    hun = _norm(un[...], und[...])
    hup = _norm(up[...], upd[...])
    zb = zb_ref[...]
    zu = zu_ref[...]
    ob = (jnp.dot(hbp, w2b[0:HID], preferred_element_type=jnp.float32)
          + jnp.dot(hbn, w2b[HID:2 * HID], preferred_element_type=jnp.float32)
          + jnp.dot(zb, w2b[2 * HID:3 * HID],
                    preferred_element_type=jnp.float32)
          + b2b[...])
    ou = (jnp.dot(hup, w2u[0:HID], preferred_element_type=jnp.float32)
          + jnp.dot(hun, w2u[HID:2 * HID], preferred_element_type=jnp.float32)
          + jnp.dot(zu, w2u[2 * HID:3 * HID],
                    preferred_element_type=jnp.float32)
          + b2u[...])
    z2b = jnp.tanh(ob)
    z2u = jnp.tanh(ou)
    o_ref[...] = jnp.tanh(
        jnp.dot(z2b, wf[0:HID], preferred_element_type=jnp.float32)
        + jnp.dot(z2u, wf[HID:D], preferred_element_type=jnp.float32)
        + bf[...])


def _tck2(bp, bn, un, up, bpd, bnd, und, upd, zb, zu,
          w2bt, b2b, w2ut, b2u, wft, bf):
    acc_spec = pl.BlockSpec((NC, BLK, HID), lambda i: (0, i, 0))
    den_spec = pl.BlockSpec((NC, NS, BLK), lambda i: (0, 0, i))
    z_spec = pl.BlockSpec((BLK, HID), lambda i: (i, 0))
    w_spec = pl.BlockSpec((3 * HID, HID), lambda i: (0, 0))
    b_spec = pl.BlockSpec((1, HID), lambda i: (0, 0))
    return pl.pallas_call(
        _tck2_body,
        grid=(NP // BLK,),
        in_specs=[acc_spec, acc_spec, acc_spec, acc_spec,
                  den_spec, den_spec, den_spec, den_spec, z_spec, z_spec,
                  w_spec, b_spec, w_spec, b_spec,
                  pl.BlockSpec((D, D), lambda i: (0, 0)),
                  pl.BlockSpec((1, D), lambda i: (0, 0))],
        out_specs=pl.BlockSpec((BLK, D), lambda i: (i, 0)),
        out_shape=jax.ShapeDtypeStruct((NP, D), jnp.float32),
    )(bp, bn, un, up, bpd, bnd, und, upd, zb, zu,
      w2bt, b2b, w2ut, b2u, wft, bf)


def _pad_edges(ei):
    s = ei[0].astype(jnp.int32)
    d = ei[1].astype(jnp.int32)
    sp = jnp.concatenate([s, jnp.zeros((EP - E,), jnp.int32)])
    dp = jnp.concatenate([d, jnp.full((EP - E,), N, jnp.int32)])
    return sp.reshape(NW, EPW), dp.reshape(NW, EPW)


def kernel(x, pos_edge_index, neg_edge_index, W1_b, b1_b, att1_b, W1_u, b1_u,
           att1_u, W2_b, b2_b, att2_bp, att2_bn, W2_u, b2_u, att2_up,
           att2_un, Wf, bf):
    f32 = jnp.float32
    x = x.astype(f32)
    spp, dpp = _pad_edges(pos_edge_index)
    snn, dnn = _pad_edges(neg_edge_index)
    x_pad = jnp.pad(x, ((0, NP - N), (0, 0)))
    x0 = x_pad[:, :HID]
    x1 = x_pad[:, HID:]

    a1 = jnp.zeros((D, 128), f32)
    a1 = a1.at[:, 0].set(att1_b[:D]).at[:, 1].set(att1_b[D:])
    a1 = a1.at[:, 2].set(att1_u[:D]).at[:, 3].set(att1_u[D:])
    al1 = _alpha1(x_pad, a1)

    pb0, pden = _sc_agg(x0, spp, dpp, al1[:, 0], al1[:, 1])
    pb1, _ = _sc_agg(x1, spp, dpp, al1[:, 0], al1[:, 1])
    nu0, nden = _sc_agg(x0, snn, dnn, al1[:, 2], al1[:, 3])
    nu1, _ = _sc_agg(x1, snn, dnn, al1[:, 2], al1[:, 3])

    a2b = jnp.zeros((HID, 128), f32)
    a2b = a2b.at[:, 0].set(att2_bp[:HID]).at[:, 1].set(att2_bp[HID:])
    a2b = a2b.at[:, 2].set(att2_un[:HID]).at[:, 3].set(att2_un[HID:])
    a2u = jnp.zeros((HID, 128), f32)
    a2u = a2u.at[:, 4].set(att2_up[:HID]).at[:, 5].set(att2_up[HID:])
    a2u = a2u.at[:, 6].set(att2_bn[:HID]).at[:, 7].set(att2_bn[HID:])

    zb, zu, al2 = _tck1(pb0, pb1, nu0, nu1, pden, nden, x_pad,
                        W1_b.T, b1_b.reshape(1, HID),
                        W1_u.T, b1_u.reshape(1, HID), a2b, a2u)

    bp, bpd = _sc_agg(zb, spp, dpp, al2[:, 0], al2[:, 1])
    un, und = _sc_agg(zb, snn, dnn, al2[:, 2], al2[:, 3])
    up, upd = _sc_agg(zu, spp, dpp, al2[:, 4], al2[:, 5])
    bn, bnd = _sc_agg(zu, snn, dnn, al2[:, 6], al2[:, 7])

    out = _tck2(bp, bn, un, up, bpd, bnd, und, upd, zb, zu,
                W2_b.T, b2_b.reshape(1, HID),
                W2_u.T, b2_u.reshape(1, HID),
                Wf.T, bf.reshape(1, D))
    return out[:N]


# speedup vs baseline: 3.4823x; 1.4019x over previous
import jax
import jax.numpy as jnp
from jax import lax
from jax.experimental import pallas as pl
from jax.experimental.pallas import tpu as pltpu
from jax.experimental.pallas import tpu_sc as plsc

N = 10000
NP = 10240
D = 256
HID = 128
E = 80000
NC, NS, L = 2, 16, 16
NW = NC * NS
CH = 64
EPW = 2560
NCHUNK = EPW // CH
EP = NW * EPW
EPS = 1e-16


HALF = NP // 2
RPH = HALF // NS
DUMMY = HALF - 1


def _sc_body(feat, src, dst, ad, asr, acc_out, den_out,
             ad_v, as_v, den_v, src_v, dst_v,
             srcA, dstA, scoA, srcB, dstB, scoB,
             grows0, grows1, acc, semg0, semg1, sems0, sems1):
    c = lax.axis_index("c")
    s = lax.axis_index("s")
    wid = s * NC + c

    pltpu.sync_copy(src.at[wid], src_v)
    pltpu.sync_copy(dst.at[wid], dst_v)
    pltpu.sync_copy(ad, ad_v)
    pltpu.sync_copy(asr, as_v)

    zero = jnp.zeros((L,), jnp.float32)
    zeroi = jnp.zeros((L,), jnp.int32)
    dum = jnp.full((L,), DUMMY, jnp.int32)

    def zden(g, carry):
        den_v[pl.ds(g * L, L)] = zero
        return carry

    lax.fori_loop(0, NP // L, zden, 0, unroll=8)

    def ibkt(j, carry):
        for k in range(CH // L):
            srcA[j, pl.ds(k * L, L)] = zeroi
            srcB[j, pl.ds(k * L, L)] = zeroi
            dstA[j, pl.ds(k * L, L)] = dum
            dstB[j, pl.ds(k * L, L)] = dum
            scoA[j, pl.ds(k * L, L)] = zero
            scoB[j, pl.ds(k * L, L)] = zero
        return carry

    lax.fori_loop(0, NCHUNK, ibkt, 0)

    def binb(g, carry):
        cA, cB = carry
        de = dst_v[pl.ds(g * L, L)]
        se = src_v[pl.ds(g * L, L)]
        t = plsc.load_gather(ad_v, [de]) + plsc.load_gather(as_v, [se])
        t = jnp.maximum(t, 0.2 * t)
        sco = jnp.exp(t)
        plsc.addupdate_scatter(den_v, [de], sco)
        m = de < HALF
        mi = m.astype(jnp.int32)
        nA = jnp.sum(mi)
        posA = cA + plsc.cumsum(mi) - 1
        mB = jnp.logical_not(m)
        posB = cB + plsc.cumsum(mB.astype(jnp.int32)) - 1
        rA, qA = posA >> 6, posA & (CH - 1)
        rB, qB = posB >> 6, posB & (CH - 1)
        plsc.store_scatter(srcA, [rA, qA], se, mask=m)
        plsc.store_scatter(dstA, [rA, qA], de, mask=m)
        plsc.store_scatter(scoA, [rA, qA], sco, mask=m)
        plsc.store_scatter(srcB, [rB, qB], se, mask=mB)
        plsc.store_scatter(dstB, [rB, qB], de - HALF, mask=mB)
        plsc.store_scatter(scoB, [rB, qB], sco, mask=mB)
        return cA + nA, cB + (L - nA)

    cA, cB = lax.fori_loop(0, EPW // L, binb, (jnp.int32(0), jnp.int32(0)))

    base = s * RPH
    for off, sb_, db_, scb_, cnt in ((0, srcA, dstA, scoA, cA),
                                     (HALF, srcB, dstB, scoB, cB)):
        def zs(r, carry):
            for k in range(HID // L):
                grows0[r, pl.ds(k * L, L)] = zero
            return carry

        lax.fori_loop(0, CH, zs, 0, unroll=4)
        for i in range(RPH // CH):
            pltpu.sync_copy(grows0, acc.at[pl.ds(base + i * CH, CH)])
        plsc.subcore_barrier()

        def scale(j, gb):
            def row(r, carry2):
                r16 = jnp.full((L,), r, jnp.int32)
                j16 = jnp.full((L,), j, jnp.int32)
                sv = plsc.load_gather(scb_, [j16, r16])
                for k in range(HID // L):
                    gb[r, pl.ds(k * L, L)] = gb[r, pl.ds(k * L, L)] * sv
                return carry2

            lax.fori_loop(0, CH, row, 0, unroll=4)

        def pair(t, carry):
            j0 = 2 * t
            j1 = j0 + 1
            g0 = pltpu.async_copy(feat.at[sb_.at[j0]], grows0, semg0)
            g1 = pltpu.async_copy(feat.at[sb_.at[j1]], grows1, semg1)
            g0.wait()
            scale(j0, grows0)
            s0 = pltpu.async_copy(grows0, acc.at[db_.at[j0]], sems0, add=True)
            g1.wait()
            scale(j1, grows1)
            s1 = pltpu.async_copy(grows1, acc.at[db_.at[j1]], sems1, add=True)
            s0.wait()
            s1.wait()
            return carry

        lax.fori_loop(0, (cnt + 2 * CH - 1) >> 7, pair, 0)
        plsc.subcore_barrier()

        pltpu.sync_copy(acc.at[pl.ds(base, RPH)],
                        acc_out.at[c, pl.ds(off + base, RPH)])

    pltpu.sync_copy(den_v, den_out.at[c, s])


_sc_agg = pl.kernel(
    _sc_body,
    out_type=(jax.ShapeDtypeStruct((NC, NP, HID), jnp.float32),
              jax.ShapeDtypeStruct((NC, NS, NP), jnp.float32)),
    mesh=plsc.VectorSubcoreMesh(core_axis_name="c", subcore_axis_name="s",
                                num_cores=NC, num_subcores=NS),
    scratch_types=[
        pltpu.VMEM((NP,), jnp.float32),
        pltpu.VMEM((NP,), jnp.float32),
        pltpu.VMEM((NP,), jnp.float32),
        pltpu.VMEM((EPW,), jnp.int32),
        pltpu.VMEM((EPW,), jnp.int32),
        pltpu.VMEM((NCHUNK, CH), jnp.int32),
        pltpu.VMEM((NCHUNK, CH), jnp.int32),
        pltpu.VMEM((NCHUNK, CH), jnp.float32),
        pltpu.VMEM((NCHUNK, CH), jnp.int32),
        pltpu.VMEM((NCHUNK, CH), jnp.int32),
        pltpu.VMEM((NCHUNK, CH), jnp.float32),
        pltpu.VMEM((CH, HID), jnp.float32),
        pltpu.VMEM((CH, HID), jnp.float32),
        pltpu.VMEM_SHARED((HALF, HID), jnp.float32),
        pltpu.SemaphoreType.DMA,
        pltpu.SemaphoreType.DMA,
        pltpu.SemaphoreType.DMA,
        pltpu.SemaphoreType.DMA,
    ],
    compiler_params=pltpu.CompilerParams(needs_layout_passes=False),
    name="snea_sc_agg",
)


BLK = 512


def _mm_body(x_ref, a_ref, o_ref):
    o_ref[...] = jnp.dot(x_ref[...], a_ref[...],
                         preferred_element_type=jnp.float32)


def _alpha1(x_pad, a1):
    return pl.pallas_call(
        _mm_body,
        grid=(NP // BLK,),
        in_specs=[
            pl.BlockSpec((BLK, D), lambda i: (i, 0)),
            pl.BlockSpec((D, 128), lambda i: (0, 0)),
        ],
        out_specs=pl.BlockSpec((BLK, 128), lambda i: (i, 0)),
        out_shape=jax.ShapeDtypeStruct((NP, 128), jnp.float32),
    )(x_pad, a1)


def _norm(acc, den):
    tot = acc[0] + acc[1]
    d = jnp.sum(den, axis=(0, 1))
    return tot * (1.0 / (d + EPS))[:, None]


def _tck1_body(pb0, pb1, nu0, nu1, pden, nden, x_ref,
               w1b, b1b, w1u, b1u, a2b, a2u, zb_o, zu_o, al_o):
    pd = pden[...]
    nd = nden[...]
    h0 = _norm(pb0[...], pd)
    h1 = _norm(pb1[...], pd)
    g0 = _norm(nu0[...], nd)
    g1 = _norm(nu1[...], nd)
    xb = x_ref[...]
    ob = (jnp.dot(h0, w1b[0:HID], preferred_element_type=jnp.float32)
          + jnp.dot(h1, w1b[HID:D], preferred_element_type=jnp.float32)
          + jnp.dot(xb, w1b[D:2 * D], preferred_element_type=jnp.float32)
          + b1b[...])
    ou = (jnp.dot(g0, w1u[0:HID], preferred_element_type=jnp.float32)
          + jnp.dot(g1, w1u[HID:D], preferred_element_type=jnp.float32)
          + jnp.dot(xb, w1u[D:2 * D], preferred_element_type=jnp.float32)
          + b1u[...])
    zb = jnp.tanh(ob)
    zu = jnp.tanh(ou)
    zb_o[...] = zb
    zu_o[...] = zu
    al_o[...] = (jnp.dot(zb, a2b[...], preferred_element_type=jnp.float32)
                 + jnp.dot(zu, a2u[...], preferred_element_type=jnp.float32))


def _tck1(pb0, pb1, nu0, nu1, pden, nden, x_pad, w1bt, b1b, w1ut, b1u,
          a2b, a2u):
    acc_spec = pl.BlockSpec((NC, BLK, HID), lambda i: (0, i, 0))
    den_spec = pl.BlockSpec((NC, NS, BLK), lambda i: (0, 0, i))
    w_spec = pl.BlockSpec((2 * D, HID), lambda i: (0, 0))
    b_spec = pl.BlockSpec((1, HID), lambda i: (0, 0))
    a_spec = pl.BlockSpec((HID, 128), lambda i: (0, 0))
    o_spec = pl.BlockSpec((BLK, 128), lambda i: (i, 0))
    return pl.pallas_call(
        _tck1_body,
        grid=(NP // BLK,),
        in_specs=[acc_spec, acc_spec, acc_spec, acc_spec, den_spec, den_spec,
                  pl.BlockSpec((BLK, D), lambda i: (i, 0)),
                  w_spec, b_spec, w_spec, b_spec, a_spec, a_spec],
        out_specs=[o_spec, o_spec, o_spec],
        out_shape=[jax.ShapeDtypeStruct((NP, 128), jnp.float32)] * 3,
    )(pb0, pb1, nu0, nu1, pden, nden, x_pad, w1bt, b1b, w1ut, b1u, a2b, a2u)


def _tck2_body(bp, bn, un, up, bpd, bnd, und, upd, zb_ref, zu_ref,
               w2b, b2b, w2u, b2u, wf, bf, o_ref):
    hbp = _norm(bp[...], bpd[...])
    hbn = _norm(bn[...], bnd[...])
    hun = _norm(un[...], und[...])
    hup = _norm(up[...], upd[...])
    zb = zb_ref[...]
    zu = zu_ref[...]
    ob = (jnp.dot(hbp, w2b[0:HID], preferred_element_type=jnp.float32)
          + jnp.dot(hbn, w2b[HID:2 * HID], preferred_element_type=jnp.float32)
          + jnp.dot(zb, w2b[2 * HID:3 * HID],
                    preferred_element_type=jnp.float32)
          + b2b[...])
    ou = (jnp.dot(hup, w2u[0:HID], preferred_element_type=jnp.float32)
          + jnp.dot(hun, w2u[HID:2 * HID], preferred_element_type=jnp.float32)
          + jnp.dot(zu, w2u[2 * HID:3 * HID],
                    preferred_element_type=jnp.float32)
          + b2u[...])
    z2b = jnp.tanh(ob)
    z2u = jnp.tanh(ou)
    o_ref[...] = jnp.tanh(
        jnp.dot(z2b, wf[0:HID], preferred_element_type=jnp.float32)
        + jnp.dot(z2u, wf[HID:D], preferred_element_type=jnp.float32)
        + bf[...])


def _tck2(bp, bn, un, up, bpd, bnd, und, upd, zb, zu,
          w2bt, b2b, w2ut, b2u, wft, bf):
    acc_spec = pl.BlockSpec((NC, BLK, HID), lambda i: (0, i, 0))
    den_spec = pl.BlockSpec((NC, NS, BLK), lambda i: (0, 0, i))
    z_spec = pl.BlockSpec((BLK, HID), lambda i: (i, 0))
    w_spec = pl.BlockSpec((3 * HID, HID), lambda i: (0, 0))
    b_spec = pl.BlockSpec((1, HID), lambda i: (0, 0))
    return pl.pallas_call(
        _tck2_body,
        grid=(NP // BLK,),
        in_specs=[acc_spec, acc_spec, acc_spec, acc_spec,
                  den_spec, den_spec, den_spec, den_spec, z_spec, z_spec,
                  w_spec, b_spec, w_spec, b_spec,
                  pl.BlockSpec((D, D), lambda i: (0, 0)),
                  pl.BlockSpec((1, D), lambda i: (0, 0))],
        out_specs=pl.BlockSpec((BLK, D), lambda i: (i, 0)),
        out_shape=jax.ShapeDtypeStruct((NP, D), jnp.float32),
    )(bp, bn, un, up, bpd, bnd, und, upd, zb, zu,
      w2bt, b2b, w2ut, b2u, wft, bf)


def _pad_edges(ei):
    s = ei[0].astype(jnp.int32)
    d = ei[1].astype(jnp.int32)
    sp = jnp.concatenate([s, jnp.zeros((EP - E,), jnp.int32)])
    dp = jnp.concatenate([d, jnp.full((EP - E,), N, jnp.int32)])
    return sp.reshape(NW, EPW), dp.reshape(NW, EPW)


def kernel(x, pos_edge_index, neg_edge_index, W1_b, b1_b, att1_b, W1_u, b1_u,
           att1_u, W2_b, b2_b, att2_bp, att2_bn, W2_u, b2_u, att2_up,
           att2_un, Wf, bf):
    f32 = jnp.float32
    x = x.astype(f32)
    spp, dpp = _pad_edges(pos_edge_index)
    snn, dnn = _pad_edges(neg_edge_index)
    x_pad = jnp.pad(x, ((0, NP - N), (0, 0)))
    x0 = x_pad[:, :HID]
    x1 = x_pad[:, HID:]

    a1 = jnp.zeros((D, 128), f32)
    a1 = a1.at[:, 0].set(att1_b[:D]).at[:, 1].set(att1_b[D:])
    a1 = a1.at[:, 2].set(att1_u[:D]).at[:, 3].set(att1_u[D:])
    al1 = _alpha1(x_pad, a1)

    pb0, pden = _sc_agg(x0, spp, dpp, al1[:, 0], al1[:, 1])
    pb1, _ = _sc_agg(x1, spp, dpp, al1[:, 0], al1[:, 1])
    nu0, nden = _sc_agg(x0, snn, dnn, al1[:, 2], al1[:, 3])
    nu1, _ = _sc_agg(x1, snn, dnn, al1[:, 2], al1[:, 3])

    a2b = jnp.zeros((HID, 128), f32)
    a2b = a2b.at[:, 0].set(att2_bp[:HID]).at[:, 1].set(att2_bp[HID:])
    a2b = a2b.at[:, 2].set(att2_un[:HID]).at[:, 3].set(att2_un[HID:])
    a2u = jnp.zeros((HID, 128), f32)
    a2u = a2u.at[:, 4].set(att2_up[:HID]).at[:, 5].set(att2_up[HID:])
    a2u = a2u.at[:, 6].set(att2_bn[:HID]).at[:, 7].set(att2_bn[HID:])

    zb, zu, al2 = _tck1(pb0, pb1, nu0, nu1, pden, nden, x_pad,
                        W1_b.T, b1_b.reshape(1, HID),
                        W1_u.T, b1_u.reshape(1, HID), a2b, a2u)

    bp, bpd = _sc_agg(zb, spp, dpp, al2[:, 0], al2[:, 1])
    un, und = _sc_agg(zb, snn, dnn, al2[:, 2], al2[:, 3])
    up, upd = _sc_agg(zu, spp, dpp, al2[:, 4], al2[:, 5])
    bn, bnd = _sc_agg(zu, snn, dnn, al2[:, 6], al2[:, 7])

    out = _tck2(bp, bn, un, up, bpd, bnd, und, upd, zb, zu,
                W2_b.T, b2_b.reshape(1, HID),
                W2_u.T, b2_u.reshape(1, HID),
                Wf.T, bf.reshape(1, D))
    return out[:N]
